# baseline (speedup 1.0000x reference)
import functools

import jax
import jax.numpy as jnp
from jax import lax
from jax.experimental import pallas as pl
from jax.experimental.pallas import tpu as pltpu
from jax.experimental.pallas import tpu_sc as plsc

B, S, G = 1024, 50, 8
D = 64
VOCAB = 1000000
N = B * S * G
BS = B * S
NC, NS = 2, 16
NW = NC * NS
PER_W = N // NW
ROWS_W = BS // NW
CHUNK = 128
NBUF = 4
SLAB = NBUF * CHUNK
NSLAB = PER_W // SLAB
OUT_SLAB = SLAB // G
LANES = 16

PKB = 4096
PHV = 123 * PKB
PGRID = PHV // PKB
RMAXB = (VOCAB - 1) // PKB

_mesh = plsc.VectorSubcoreMesh(core_axis_name="c", subcore_axis_name="s")


def _pack_body(l_ref, r_ref, o_ref):
    for t in range(PKB // 128):
        sl = pl.ds(t * 128, 128)
        z = jnp.concatenate([l_ref[:, sl], r_ref[:, sl]], axis=0)
        o_ref[sl, :] = z.T.astype(jnp.bfloat16)


_pack_tc = pl.pallas_call(
    _pack_body,
    grid=(PGRID,),
    in_specs=[
        pl.BlockSpec((D, PKB), lambda b: (0, b)),
        pl.BlockSpec((D, PKB), lambda b: (0, jnp.minimum(PGRID + b, RMAXB))),
    ],
    out_specs=pl.BlockSpec((PKB, 2 * D), lambda b: (b, 0)),
    out_shape=jax.ShapeDtypeStruct((PHV, 2 * D), jnp.bfloat16),
)


@functools.partial(
    pl.kernel,
    mesh=_mesh,
    compiler_params=pltpu.CompilerParams(
        use_tc_tiling_on_sc=False, needs_layout_passes=False),
    out_type=jax.ShapeDtypeStruct((BS // 2, 2 * D), jnp.float32),
    scratch_types=[
        pltpu.VMEM((PER_W,), jnp.int32),
        pltpu.VMEM((PER_W,), jnp.int32),
        pltpu.VMEM((PER_W,), jnp.float32),
        pltpu.VMEM((ROWS_W,), jnp.float32),
        pltpu.VMEM((CHUNK, 2 * D), jnp.bfloat16),
        pltpu.VMEM((CHUNK, 2 * D), jnp.bfloat16),
        pltpu.VMEM((CHUNK, 2 * D), jnp.bfloat16),
        pltpu.VMEM((CHUNK, 2 * D), jnp.bfloat16),
        pltpu.VMEM((OUT_SLAB // 2, 2 * D), jnp.float32),
        pltpu.SemaphoreType.DMA,
        pltpu.SemaphoreType.DMA,
        pltpu.SemaphoreType.DMA,
        pltpu.SemaphoreType.DMA,
    ],
)
def _gaz_embed_sc(idx_hbm, mask_hbm, len_hbm, tbl_hbm, out_hbm,
                  idx_v, col_v, mask_v, len_v, rv0, rv1, rv2, rv3,
                  out_v, sem0, sem1, sem2, sem3):
    rows_bufs = (rv0, rv1, rv2, rv3)
    sems = (sem0, sem1, sem2, sem3)
    wid = lax.axis_index("s") * NC + lax.axis_index("c")
    ibase = wid * PER_W
    rbase = wid * ROWS_W
    pltpu.sync_copy(idx_hbm.at[pl.ds(ibase, PER_W)], idx_v)
    pltpu.sync_copy(mask_hbm.at[pl.ds(ibase, PER_W)], mask_v)
    pltpu.sync_copy(len_hbm.at[pl.ds(rbase, ROWS_W)], len_v)

    def prep_body(t, _):
        sl = pl.ds(t * LANES, LANES)
        v = idx_v[sl]
        ge = v >= PHV
        idx_v[sl] = v - jnp.where(ge, PHV, 0)
        col_v[sl] = jnp.where(ge, D, 0)
        return 0

    lax.fori_loop(0, PER_W // LANES, prep_body, 0)

    def gather(chunk_off, buf, sem):
        return pltpu.async_copy(
            tbl_hbm.at[idx_v.at[pl.ds(chunk_off, CHUNK)]], buf, sem)

    for b in range(NBUF):
        gather(b * CHUNK, rows_bufs[b], sems[b])

    def slab_body(s_i, _):
        soff = s_i * SLAB
        for b in range(NBUF):
            coff = soff + b * CHUNK
            rows_v = rows_bufs[b]
            pltpu.make_async_copy(
                tbl_hbm.at[idx_v.at[pl.ds(coff, CHUNK)]], rows_v, sems[b]
            ).wait()
            obase = b * (CHUNK // G)
            inv_vec = 1.0 / len_v[pl.ds(s_i * OUT_SLAB + obase, LANES)]
            iota2 = 2 * lax.iota(jnp.int32, LANES)
            for half in range(CHUNK // LANES):
                mv = mask_v[pl.ds(coff + half * LANES, LANES)]
                cv = col_v[pl.ds(coff + half * LANES, LANES)]
                for sub in range(2):
                    r = half * 2 + sub
                    r0 = r * G
                    inv = inv_vec[r]
                    opack = (obase + r) // 2
                    ocol = (r % 2) * D
                    for h in range(2):
                        ae = None
                        ao = None
                        for g in range(G):
                            m = mv[sub * G + g]
                            x = rows_v[r0 + g, pl.ds(cv[sub * G + g] + h * 2 * LANES, 2 * LANES)]
                            xi = plsc.bitcast(x, jnp.int32)
                            fe = plsc.bitcast(xi << 16, jnp.float32) * m
                            fo = plsc.bitcast(xi & jnp.int32(-65536), jnp.float32) * m
                            ae = fe if ae is None else ae + fe
                            ao = fo if ao is None else ao + fo
                        rsp = jnp.full((LANES,), opack, jnp.int32)
                        cbase = ocol + h * 2 * LANES
                        plsc.store_scatter(out_v, [rsp, cbase + iota2], ae * inv)
                        plsc.store_scatter(out_v, [rsp, cbase + 1 + iota2], ao * inv)
            @pl.when(s_i < NSLAB - 1)
            def _():
                gather(coff + SLAB, rows_v, sems[b])

        pltpu.sync_copy(
            out_v,
            out_hbm.at[pl.ds((rbase + s_i * OUT_SLAB) // 2, OUT_SLAB // 2)],
        )
        return 0

    lax.fori_loop(0, NSLAB, slab_body, 0)


def kernel(gaz_seq_tensor, gaz_seq_lengths, gaz_mask_tensor, gaz_embedding):
    idx = gaz_seq_tensor.reshape(N).astype(jnp.int32)
    mask = gaz_mask_tensor.reshape(N)
    lens = gaz_seq_lengths.reshape(BS).astype(jnp.float32)
    tbl_t = gaz_embedding.T
    tbl2 = _pack_tc(tbl_t, tbl_t)
    out = _gaz_embed_sc(idx, mask, lens, tbl2)
    return out.reshape(B, S, D)


# speedup vs baseline: 1.5278x; 1.5278x over previous
import functools

import jax
import jax.numpy as jnp
from jax import lax
from jax.experimental import pallas as pl
from jax.experimental.pallas import tpu as pltpu
from jax.experimental.pallas import tpu_sc as plsc

B, S, G = 1024, 50, 8
D = 64
VOCAB = 1000000
N = B * S * G
BS = B * S
NC, NS = 2, 16
NW = NC * NS
PER_W = N // NW
ROWS_W = BS // NW
CHUNK = 128
NBUF = 4
SLAB = NBUF * CHUNK
NSLAB = PER_W // SLAB
OUT_SLAB = SLAB // G
LANES = 16

PKB = 4096
PHV = 123 * PKB
PGRID = PHV // PKB
RMAXB = (VOCAB - 1) // PKB

_mesh = plsc.VectorSubcoreMesh(core_axis_name="c", subcore_axis_name="s")


def _pack_body(l_ref, r_ref, o_ref):
    for t in range(PKB // 128):
        sl = pl.ds(t * 128, 128)
        z = jnp.concatenate([l_ref[:, sl], r_ref[:, sl]], axis=0)
        o_ref[sl, :] = z.T


_pack_tc = pl.pallas_call(
    _pack_body,
    grid=(PGRID,),
    in_specs=[
        pl.BlockSpec((D, PKB), lambda b: (0, b)),
        pl.BlockSpec((D, PKB), lambda b: (0, jnp.minimum(PGRID + b, RMAXB))),
    ],
    out_specs=pl.BlockSpec((PKB, 2 * D), lambda b: (b, 0)),
    out_shape=jax.ShapeDtypeStruct((PHV, 2 * D), jnp.float32),
)


@functools.partial(
    pl.kernel,
    mesh=_mesh,
    compiler_params=pltpu.CompilerParams(
        use_tc_tiling_on_sc=False, needs_layout_passes=False),
    out_type=jax.ShapeDtypeStruct((BS // 2, 2 * D), jnp.float32),
    scratch_types=[
        pltpu.VMEM((PER_W,), jnp.int32),
        pltpu.VMEM((PER_W,), jnp.int32),
        pltpu.VMEM((PER_W,), jnp.float32),
        pltpu.VMEM((ROWS_W,), jnp.float32),
        pltpu.VMEM((CHUNK, 2 * D), jnp.float32),
        pltpu.VMEM((CHUNK, 2 * D), jnp.float32),
        pltpu.VMEM((CHUNK, 2 * D), jnp.float32),
        pltpu.VMEM((CHUNK, 2 * D), jnp.float32),
        pltpu.VMEM((OUT_SLAB // 2, 2 * D), jnp.float32),
        pltpu.SemaphoreType.DMA,
        pltpu.SemaphoreType.DMA,
        pltpu.SemaphoreType.DMA,
        pltpu.SemaphoreType.DMA,
    ],
)
def _gaz_embed_sc(idx_hbm, mask_hbm, len_hbm, tbl_hbm, out_hbm,
                  idx_v, col_v, mask_v, len_v, rv0, rv1, rv2, rv3,
                  out_v, sem0, sem1, sem2, sem3):
    rows_bufs = (rv0, rv1, rv2, rv3)
    sems = (sem0, sem1, sem2, sem3)
    wid = lax.axis_index("s") * NC + lax.axis_index("c")
    ibase = wid * PER_W
    rbase = wid * ROWS_W
    pltpu.sync_copy(idx_hbm.at[pl.ds(ibase, PER_W)], idx_v)
    pltpu.sync_copy(mask_hbm.at[pl.ds(ibase, PER_W)], mask_v)
    pltpu.sync_copy(len_hbm.at[pl.ds(rbase, ROWS_W)], len_v)

    def prep_body(t, _):
        sl = pl.ds(t * LANES, LANES)
        v = idx_v[sl]
        ge = v >= PHV
        idx_v[sl] = v - jnp.where(ge, PHV, 0)
        col_v[sl] = jnp.where(ge, D, 0)
        return 0

    lax.fori_loop(0, PER_W // LANES, prep_body, 0)

    def gather(chunk_off, buf, sem):
        return pltpu.async_copy(
            tbl_hbm.at[idx_v.at[pl.ds(chunk_off, CHUNK)]], buf, sem)

    for b in range(NBUF):
        gather(b * CHUNK, rows_bufs[b], sems[b])

    def slab_body(s_i, _):
        soff = s_i * SLAB
        for b in range(NBUF):
            coff = soff + b * CHUNK
            rows_v = rows_bufs[b]
            pltpu.make_async_copy(
                tbl_hbm.at[idx_v.at[pl.ds(coff, CHUNK)]], rows_v, sems[b]
            ).wait()
            obase = b * (CHUNK // G)
            inv_vec = 1.0 / len_v[pl.ds(s_i * OUT_SLAB + obase, LANES)]
            for half in range(CHUNK // LANES):
                mv = mask_v[pl.ds(coff + half * LANES, LANES)]
                cv = col_v[pl.ds(coff + half * LANES, LANES)]
                for sub in range(2):
                    r = half * 2 + sub
                    r0 = r * G
                    inv = inv_vec[r]
                    opack = (obase + r) // 2
                    ocol = (r % 2) * D
                    for d_blk in range(D // LANES):
                        dof = d_blk * LANES
                        acc = rows_v[r0, pl.ds(cv[sub * G] + dof, LANES)] * mv[sub * G]
                        for g in range(1, G):
                            acc = acc + rows_v[r0 + g, pl.ds(cv[sub * G + g] + dof, LANES)] * mv[sub * G + g]
                        out_v[opack, pl.ds(ocol + dof, LANES)] = acc * inv
            @pl.when(s_i < NSLAB - 1)
            def _():
                gather(coff + SLAB, rows_v, sems[b])

        pltpu.sync_copy(
            out_v,
            out_hbm.at[pl.ds((rbase + s_i * OUT_SLAB) // 2, OUT_SLAB // 2)],
        )
        return 0

    lax.fori_loop(0, NSLAB, slab_body, 0)


def kernel(gaz_seq_tensor, gaz_seq_lengths, gaz_mask_tensor, gaz_embedding):
    idx = gaz_seq_tensor.reshape(N).astype(jnp.int32)
    mask = gaz_mask_tensor.reshape(N)
    lens = gaz_seq_lengths.reshape(BS).astype(jnp.float32)
    tbl_t = gaz_embedding.T
    tbl2 = _pack_tc(tbl_t, tbl_t)
    out = _gaz_embed_sc(idx, mask, lens, tbl2)
    return out.reshape(B, S, D)


# speedup vs baseline: 2.1086x; 1.3802x over previous
import functools

import jax
import jax.numpy as jnp
from jax import lax
from jax.experimental import pallas as pl
from jax.experimental.pallas import tpu as pltpu
from jax.experimental.pallas import tpu_sc as plsc

B, S, G = 1024, 50, 8
D = 64
VOCAB = 1000000
N = B * S * G
BS = B * S
NC, NS = 2, 16
NW = NC * NS
PER_W = N // NW
ROWS_W = BS // NW
CHUNK = 128
NBUF = 4
SLAB = NBUF * CHUNK
NSLAB = PER_W // SLAB
OUT_SLAB = SLAB // G
LANES = 16

PKB = 4096
PHV = 123 * PKB
PGRID = PHV // PKB
RMAXB = (VOCAB - 1) // PKB

_mesh = plsc.VectorSubcoreMesh(core_axis_name="c", subcore_axis_name="s")


def _pack_body(l_ref, r_ref, o_ref):
    for t in range(PKB // 128):
        sl = pl.ds(t * 128, 128)
        z = jnp.concatenate([l_ref[:, sl], r_ref[:, sl]], axis=0)
        o_ref[sl, :] = z.T


_pack_tc = pl.pallas_call(
    _pack_body,
    grid=(PGRID,),
    in_specs=[
        pl.BlockSpec((D, PKB), lambda b: (0, b)),
        pl.BlockSpec((D, PKB), lambda b: (0, jnp.minimum(PGRID + b, RMAXB))),
    ],
    out_specs=pl.BlockSpec((PKB, 2 * D), lambda b: (b, 0)),
    out_shape=jax.ShapeDtypeStruct((PHV, 2 * D), jnp.float32),
)


@functools.partial(
    pl.kernel,
    mesh=_mesh,
    compiler_params=pltpu.CompilerParams(
        use_tc_tiling_on_sc=False, needs_layout_passes=False),
    out_type=jax.ShapeDtypeStruct((BS // 2, 2 * D), jnp.float32),
    scratch_types=[
        pltpu.VMEM((PER_W,), jnp.int32),
        pltpu.VMEM((PER_W,), jnp.float32),
        pltpu.VMEM((ROWS_W,), jnp.float32),
        pltpu.VMEM((CHUNK, D), jnp.float32),
        pltpu.VMEM((CHUNK, D), jnp.float32),
        pltpu.VMEM((CHUNK, D), jnp.float32),
        pltpu.VMEM((CHUNK, D), jnp.float32),
        pltpu.VMEM((OUT_SLAB // 2, 2 * D), jnp.float32),
        pltpu.SemaphoreType.DMA,
        pltpu.SemaphoreType.DMA,
        pltpu.SemaphoreType.DMA,
        pltpu.SemaphoreType.DMA,
    ],
)
def _gaz_embed_sc(idx_hbm, mask_hbm, len_hbm, tbl_hbm, out_hbm,
                  idx_v, mask_v, len_v, rv0, rv1, rv2, rv3,
                  out_v, sem0, sem1, sem2, sem3):
    rows_bufs = (rv0, rv1, rv2, rv3)
    sems = (sem0, sem1, sem2, sem3)
    wid = lax.axis_index("s") * NC + lax.axis_index("c")
    ibase = wid * PER_W
    rbase = wid * ROWS_W
    pltpu.sync_copy(idx_hbm.at[pl.ds(ibase, PER_W)], idx_v)
    pltpu.sync_copy(mask_hbm.at[pl.ds(ibase, PER_W)], mask_v)
    pltpu.sync_copy(len_hbm.at[pl.ds(rbase, ROWS_W)], len_v)

    def prep_body(t, _):
        sl = pl.ds(t * LANES, LANES)
        v = idx_v[sl]
        idx_v[sl] = 2 * v - jnp.where(v >= PHV, 2 * PHV - 1, 0)
        return 0

    lax.fori_loop(0, PER_W // LANES, prep_body, 0)

    def gather(chunk_off, buf, sem):
        return pltpu.async_copy(
            tbl_hbm.at[idx_v.at[pl.ds(chunk_off, CHUNK)]], buf, sem)

    for b in range(NBUF):
        gather(b * CHUNK, rows_bufs[b], sems[b])

    def slab_body(s_i, _):
        soff = s_i * SLAB
        for b in range(NBUF):
            coff = soff + b * CHUNK
            rows_v = rows_bufs[b]
            pltpu.make_async_copy(
                tbl_hbm.at[idx_v.at[pl.ds(coff, CHUNK)]], rows_v, sems[b]
            ).wait()
            obase = b * (CHUNK // G)
            inv_vec = 1.0 / len_v[pl.ds(s_i * OUT_SLAB + obase, LANES)]
            for half in range(CHUNK // LANES):
                mv = mask_v[pl.ds(coff + half * LANES, LANES)]
                for sub in range(2):
                    r = half * 2 + sub
                    r0 = r * G
                    inv = inv_vec[r]
                    opack = (obase + r) // 2
                    ocol = (r % 2) * D
                    for d_blk in range(D // LANES):
                        dof = d_blk * LANES
                        acc = rows_v[r0, pl.ds(dof, LANES)] * mv[sub * G]
                        for g in range(1, G):
                            acc = acc + rows_v[r0 + g, pl.ds(dof, LANES)] * mv[sub * G + g]
                        out_v[opack, pl.ds(ocol + dof, LANES)] = acc * inv
            @pl.when(s_i < NSLAB - 1)
            def _():
                gather(coff + SLAB, rows_v, sems[b])

        pltpu.sync_copy(
            out_v,
            out_hbm.at[pl.ds((rbase + s_i * OUT_SLAB) // 2, OUT_SLAB // 2)],
        )
        return 0

    lax.fori_loop(0, NSLAB, slab_body, 0)


def kernel(gaz_seq_tensor, gaz_seq_lengths, gaz_mask_tensor, gaz_embedding):
    idx = gaz_seq_tensor.reshape(N).astype(jnp.int32)
    mask = gaz_mask_tensor.reshape(N)
    lens = gaz_seq_lengths.reshape(BS).astype(jnp.float32)
    tbl_t = gaz_embedding.T
    tbl2 = _pack_tc(tbl_t, tbl_t)
    out = _gaz_embed_sc(idx, mask, lens, tbl2.reshape(2 * PHV, D))
    return out.reshape(B, S, D)


# speedup vs baseline: 2.2543x; 1.0691x over previous
import functools

import jax
import jax.numpy as jnp
from jax import lax
from jax.experimental import pallas as pl
from jax.experimental.pallas import tpu as pltpu
from jax.experimental.pallas import tpu_sc as plsc

B, S, G = 1024, 50, 8
D = 64
VOCAB = 1000000
N = B * S * G
BS = B * S
NC, NS = 2, 16
NW = NC * NS
PER_W = N // NW
ROWS_W = BS // NW
CHUNK = 128
NBUF = 4
SLAB = NBUF * CHUNK
NSLAB = PER_W // SLAB
OUT_SLAB = SLAB // G
LANES = 16

PKB = 8192
PHV = 62 * PKB
PGRID = PHV // PKB
RMAXB = (VOCAB - 1) // PKB

_mesh = plsc.VectorSubcoreMesh(core_axis_name="c", subcore_axis_name="s")


def _pack_body(l_ref, r_ref, o_ref):
    for t in range(PKB // 128):
        sl = pl.ds(t * 128, 128)
        z = jnp.concatenate([l_ref[:, sl], r_ref[:, sl]], axis=0)
        o_ref[sl, :] = z.T


_pack_tc = pl.pallas_call(
    _pack_body,
    grid=(PGRID,),
    in_specs=[
        pl.BlockSpec((D, PKB), lambda b: (0, b)),
        pl.BlockSpec((D, PKB), lambda b: (0, jnp.minimum(PGRID + b, RMAXB))),
    ],
    out_specs=pl.BlockSpec((PKB, 2 * D), lambda b: (b, 0)),
    out_shape=jax.ShapeDtypeStruct((PHV, 2 * D), jnp.float32),
)


@functools.partial(
    pl.kernel,
    mesh=_mesh,
    compiler_params=pltpu.CompilerParams(
        use_tc_tiling_on_sc=False, needs_layout_passes=False),
    out_type=jax.ShapeDtypeStruct((BS // 2, 2 * D), jnp.float32),
    scratch_types=[
        pltpu.VMEM((PER_W,), jnp.int32),
        pltpu.VMEM((PER_W,), jnp.float32),
        pltpu.VMEM((ROWS_W,), jnp.float32),
        pltpu.VMEM((CHUNK, D), jnp.float32),
        pltpu.VMEM((CHUNK, D), jnp.float32),
        pltpu.VMEM((CHUNK, D), jnp.float32),
        pltpu.VMEM((CHUNK, D), jnp.float32),
        pltpu.VMEM((OUT_SLAB // 2, 2 * D), jnp.float32),
        pltpu.SemaphoreType.DMA,
        pltpu.SemaphoreType.DMA,
        pltpu.SemaphoreType.DMA,
        pltpu.SemaphoreType.DMA,
    ],
)
def _gaz_embed_sc(idx_hbm, mask_hbm, len_hbm, tbl_hbm, out_hbm,
                  idx_v, mask_v, len_v, rv0, rv1, rv2, rv3,
                  out_v, sem0, sem1, sem2, sem3):
    rows_bufs = (rv0, rv1, rv2, rv3)
    sems = (sem0, sem1, sem2, sem3)
    wid = lax.axis_index("s") * NC + lax.axis_index("c")
    ibase = wid * PER_W
    rbase = wid * ROWS_W
    pltpu.sync_copy(idx_hbm.at[pl.ds(ibase, PER_W)], idx_v)
    pltpu.sync_copy(mask_hbm.at[pl.ds(ibase, PER_W)], mask_v)
    pltpu.sync_copy(len_hbm.at[pl.ds(rbase, ROWS_W)], len_v)

    def prep_body(t, _):
        sl = pl.ds(t * LANES, LANES)
        v = idx_v[sl]
        idx_v[sl] = 2 * v - jnp.where(v >= PHV, 2 * PHV - 1, 0)
        return 0

    lax.fori_loop(0, PER_W // LANES, prep_body, 0)

    def gather(chunk_off, buf, sem):
        return pltpu.async_copy(
            tbl_hbm.at[idx_v.at[pl.ds(chunk_off, CHUNK)]], buf, sem)

    for b in range(NBUF):
        gather(b * CHUNK, rows_bufs[b], sems[b])

    def slab_body(s_i, _):
        soff = s_i * SLAB
        for b in range(NBUF):
            coff = soff + b * CHUNK
            rows_v = rows_bufs[b]
            pltpu.make_async_copy(
                tbl_hbm.at[idx_v.at[pl.ds(coff, CHUNK)]], rows_v, sems[b]
            ).wait()
            obase = b * (CHUNK // G)
            inv_vec = 1.0 / len_v[pl.ds(s_i * OUT_SLAB + obase, LANES)]
            for half in range(CHUNK // LANES):
                mv = mask_v[pl.ds(coff + half * LANES, LANES)]
                for sub in range(2):
                    r = half * 2 + sub
                    r0 = r * G
                    inv = inv_vec[r]
                    opack = (obase + r) // 2
                    ocol = (r % 2) * D
                    for d_blk in range(D // LANES):
                        dof = d_blk * LANES
                        acc = rows_v[r0, pl.ds(dof, LANES)] * mv[sub * G]
                        for g in range(1, G):
                            acc = acc + rows_v[r0 + g, pl.ds(dof, LANES)] * mv[sub * G + g]
                        out_v[opack, pl.ds(ocol + dof, LANES)] = acc * inv
            @pl.when(s_i < NSLAB - 1)
            def _():
                gather(coff + SLAB, rows_v, sems[b])

        pltpu.sync_copy(
            out_v,
            out_hbm.at[pl.ds((rbase + s_i * OUT_SLAB) // 2, OUT_SLAB // 2)],
        )
        return 0

    lax.fori_loop(0, NSLAB, slab_body, 0)


def kernel(gaz_seq_tensor, gaz_seq_lengths, gaz_mask_tensor, gaz_embedding):
    idx = gaz_seq_tensor.reshape(N).astype(jnp.int32)
    mask = gaz_mask_tensor.reshape(N)
    lens = gaz_seq_lengths.reshape(BS).astype(jnp.float32)
    tbl_t = gaz_embedding.T
    tbl2 = _pack_tc(tbl_t, tbl_t)
    out = _gaz_embed_sc(idx, mask, lens, tbl2.reshape(2 * PHV, D))
    return out.reshape(B, S, D)
